# Initial kernel scaffold; baseline (speedup 1.0000x reference)
import jax
import jax.numpy as jnp
from jax.experimental import pallas as pl


def kernel(x, edge_index, edge_type, params):
    raise NotImplementedError("write your pallas kernel here")


# speedup vs baseline: 3.1446x; 3.1446x over previous
import functools

import jax
import jax.numpy as jnp
from jax import lax
from jax.experimental import pallas as pl
from jax.experimental.pallas import tpu as pltpu
from jax.experimental.pallas import tpu_sc as plsc

NC = 2
NS = 16
GCH = 128
WIN = 1024


def _make_agg(n_pad, d, e, rel):
    eps = e // NS
    cap = ((eps + GCH - 1) // GCH) * GCH + GCH
    half = n_pad // NC
    nwin = half // WIN
    rpt = WIN // NS
    mesh = plsc.VectorSubcoreMesh(
        core_axis_name="c", subcore_axis_name="s", num_cores=NC, num_subcores=NS
    )

    @functools.partial(
        pl.kernel,
        out_type=jax.ShapeDtypeStruct((n_pad, d), jnp.float32),
        mesh=mesh,
        scratch_types=[
            pltpu.VMEM((eps,), jnp.int32),
            pltpu.VMEM((eps,), jnp.int32),
            pltpu.VMEM((eps,), jnp.int32),
            pltpu.VMEM((cap,), jnp.int32),
            pltpu.VMEM((cap,), jnp.int32),
            pltpu.VMEM((GCH,), jnp.int32),
            pltpu.VMEM((GCH,), jnp.int32),
            pltpu.VMEM((GCH, d), jnp.float32),
            pltpu.VMEM_SHARED((WIN + 16, d), jnp.float32),
            pltpu.SemaphoreType.DMA,
        ],
        compiler_params=pltpu.CompilerParams(needs_layout_passes=False,
                                             use_tc_tiling_on_sc=False),
    )
    def k(h_hbm, src_hbm, dst_hbm, typ_hbm, out_hbm,
          src_raw, dst_raw, typ_raw, src_c, dst_c,
          src_stage, dst_stage, rows, agg, sem):
        c = lax.axis_index("c")
        s = lax.axis_index("s")
        base = s * eps

        pltpu.sync_copy(src_hbm.at[pl.ds(base, eps)], src_raw)
        pltpu.sync_copy(dst_hbm.at[pl.ds(base, eps)], dst_raw)
        pltpu.sync_copy(typ_hbm.at[pl.ds(base, eps)], typ_raw)

        zvec = jnp.zeros((16,), jnp.float32)

        def zrow(i, carry):
            for kk in range(d // 16):
                rows[i, pl.ds(kk * 16, 16)] = zvec
            return carry

        lax.fori_loop(0, GCH, zrow, 0)

        dummy = jnp.full((16,), WIN, jnp.int32)
        zidx = jnp.zeros((16,), jnp.int32)

        def prefill(i, carry):
            src_c[pl.ds(i * 16, 16)] = dummy
            dst_c[pl.ds(i * 16, 16)] = zidx
            return carry

        lax.fori_loop(0, cap // 16, prefill, 0)

        for w in range(nwin):
            lo = c * half + w * WIN

            pltpu.sync_copy(rows.at[pl.ds(0, rpt)], agg.at[pl.ds(s * rpt, rpt)])

            def compact(i, cnt):
                t = typ_raw[pl.ds(i * 16, 16)]
                sv = src_raw[pl.ds(i * 16, 16)] - lo
                dv = dst_raw[pl.ds(i * 16, 16)]
                m = (t == rel) & (sv >= 0) & (sv < WIN)
                inc = plsc.cumsum(m.astype(jnp.int32))
                pos = cnt + inc - 1
                plsc.store_scatter(src_c, [pos], sv, mask=m)
                plsc.store_scatter(dst_c, [pos], dv, mask=m)
                return cnt + jnp.sum(m.astype(jnp.int32))

            cnt = lax.fori_loop(0, eps // 16, compact, jnp.int32(0))

            plsc.subcore_barrier()

            nch = (cnt + GCH - 1) // GCH

            def gather_scatter(t, carry):
                off = t * GCH
                for kk in range(GCH // 16):
                    dst_stage[pl.ds(kk * 16, 16)] = dst_c[pl.ds(off + kk * 16, 16)]
                    src_stage[pl.ds(kk * 16, 16)] = src_c[pl.ds(off + kk * 16, 16)]
                pltpu.async_copy(h_hbm.at[dst_stage], rows, sem).wait()
                pltpu.sync_copy(rows, agg.at[src_stage], add=True)
                return carry

            lax.fori_loop(0, nch, gather_scatter, 0)

            ndum = nch * GCH

            def redummy(i, carry):
                src_c[pl.ds(i * 16, 16)] = dummy
                dst_c[pl.ds(i * 16, 16)] = zidx
                return carry

            lax.fori_loop(0, (ndum + 15) // 16, redummy, 0)

            lax.fori_loop(0, GCH, zrow, 0)

            plsc.subcore_barrier()
            pltpu.sync_copy(agg.at[pl.ds(s * rpt, rpt)],
                            out_hbm.at[pl.ds(lo + s * rpt, rpt)])

    return k


def _dense(agg, h, wl_t, wc_t, bias8):
    n_pad, d = h.shape
    hid = wl_t.shape[1]
    bm = 1024

    def body(a_ref, h_ref, wl_ref, wc_ref, b_ref, o_ref):
        acc = jnp.dot(a_ref[...], wl_ref[...], preferred_element_type=jnp.float32)
        acc = acc + jnp.dot(h_ref[...], wc_ref[...],
                            preferred_element_type=jnp.float32)
        o_ref[...] = jnp.maximum(acc + b_ref[0:1, :], 0.0)

    return pl.pallas_call(
        body,
        grid=(n_pad // bm,),
        in_specs=[
            pl.BlockSpec((bm, d), lambda i: (i, 0)),
            pl.BlockSpec((bm, d), lambda i: (i, 0)),
            pl.BlockSpec((d, hid), lambda i: (0, 0)),
            pl.BlockSpec((d, hid), lambda i: (0, 0)),
            pl.BlockSpec((8, hid), lambda i: (0, 0)),
        ],
        out_specs=pl.BlockSpec((bm, hid), lambda i: (i, 0)),
        out_shape=jax.ShapeDtypeStruct((n_pad, hid), jnp.float32),
    )(agg, h, wl_t, wc_t, bias8)


def kernel(x, edge_index, edge_type, params):
    n, d0 = x.shape
    e = edge_index.shape[1]
    n_pad = ((n // 2048) + 1) * 2048

    src = edge_index[0]
    dst = edge_index[1]
    h = jnp.zeros((n_pad, d0), x.dtype).at[:n].set(x)

    for rel, p in enumerate(params):
        d = h.shape[1]
        agg = _make_agg(n_pad, d, e, rel)(h, src, dst, edge_type)
        wl_t = p["w_l_W"].T
        wc_t = (p["w_0_W"] + p["w_1_W"]).T
        bias = p["w_l_b"] + p["w_0_b"] + p["w_1_b"]
        bias8 = jnp.broadcast_to(bias[None, :], (8, bias.shape[0]))
        h = _dense(agg, h, wl_t, wc_t, bias8)

    return h[:n]


# speedup vs baseline: 3.9716x; 1.2630x over previous
import functools

import jax
import jax.numpy as jnp
from jax import lax
from jax.experimental import pallas as pl
from jax.experimental.pallas import tpu as pltpu
from jax.experimental.pallas import tpu_sc as plsc

NC = 2
NS = 16
GCH = 128
PKB = 14


def _make_agg(n_pad, d, e, rel, win):
    eps = e // NS
    cap = eps + 2 * GCH
    half = n_pad // NC
    nwin = half // win
    rpt = win // NS
    mesh = plsc.VectorSubcoreMesh(
        core_axis_name="c", subcore_axis_name="s", num_cores=NC, num_subcores=NS
    )

    @functools.partial(
        pl.kernel,
        out_type=jax.ShapeDtypeStruct((n_pad, d), jnp.float32),
        mesh=mesh,
        scratch_types=[
            pltpu.VMEM((eps,), jnp.int32),
            pltpu.VMEM((eps,), jnp.int32),
            pltpu.VMEM((eps,), jnp.int32),
            pltpu.VMEM((cap,), jnp.int32),
            pltpu.VMEM((GCH,), jnp.int32),
            pltpu.VMEM((GCH,), jnp.int32),
            pltpu.VMEM((GCH,), jnp.int32),
            pltpu.VMEM((GCH,), jnp.int32),
            pltpu.VMEM((GCH, d), jnp.float32),
            pltpu.VMEM((GCH, d), jnp.float32),
            pltpu.VMEM((rpt, d), jnp.float32),
            pltpu.VMEM_SHARED((win + 16, d), jnp.float32),
            pltpu.SemaphoreType.DMA,
            pltpu.SemaphoreType.DMA,
            pltpu.SemaphoreType.DMA,
            pltpu.SemaphoreType.DMA,
        ],
        compiler_params=pltpu.CompilerParams(needs_layout_passes=False,
                                             use_tc_tiling_on_sc=False),
    )
    def k(h_hbm, src_hbm, dst_hbm, typ_hbm, out_hbm,
          src_raw, dst_raw, typ_raw, pk_c,
          src_st0, dst_st0, src_st1, dst_st1, rows0, rows1, zbuf, agg,
          sg0, sg1, ss0, ss1):
        c = lax.axis_index("c")
        s = lax.axis_index("s")
        base = s * eps

        pltpu.sync_copy(src_hbm.at[pl.ds(base, eps)], src_raw)
        pltpu.sync_copy(dst_hbm.at[pl.ds(base, eps)], dst_raw)
        pltpu.sync_copy(typ_hbm.at[pl.ds(base, eps)], typ_raw)

        zvec = jnp.zeros((16,), jnp.float32)

        def zrow(i, carry):
            for kk in range(d // 16):
                zbuf[i, pl.ds(kk * 16, 16)] = zvec
            return carry

        lax.fori_loop(0, rpt, zrow, 0)

        iota16 = lax.broadcasted_iota(jnp.int32, (16,), 0)
        dummy = jnp.full((16,), win << PKB, jnp.int32)

        def stage(off, src_st, dst_st):
            for kk in range(GCH // 16):
                v = pk_c[pl.ds(off + kk * 16, 16)]
                src_st[pl.ds(kk * 16, 16)] = lax.shift_right_logical(v, PKB)
                dst_st[pl.ds(kk * 16, 16)] = v & ((1 << PKB) - 1)

        for w in range(nwin):
            lo = c * half + w * win

            pltpu.sync_copy(zbuf, agg.at[pl.ds(s * rpt, rpt)])

            def compact(i, cnt):
                t = typ_raw[pl.ds(i * 16, 16)]
                sv = src_raw[pl.ds(i * 16, 16)] - lo
                dv = dst_raw[pl.ds(i * 16, 16)]
                m = (t == rel) & (sv >= 0) & (sv < win)
                pk = lax.shift_left(sv, PKB) | dv
                inc = plsc.cumsum(m.astype(jnp.int32))
                plsc.store_scatter(pk_c, [cnt + inc - 1], pk, mask=m)
                return cnt + jnp.sum(m.astype(jnp.int32))

            cnt = lax.fori_loop(0, eps // 16, compact, jnp.int32(0))

            for i in range(2 * GCH // 16):
                plsc.store_scatter(pk_c, [cnt + i * 16 + iota16], dummy)

            plsc.subcore_barrier()

            npair = (cnt + 2 * GCH - 1) // (2 * GCH)

            def pair(kk, carry):
                off = kk * (2 * GCH)
                stage(off, src_st0, dst_st0)
                g0 = pltpu.async_copy(h_hbm.at[dst_st0], rows0, sg0)
                stage(off + GCH, src_st1, dst_st1)
                g1 = pltpu.async_copy(h_hbm.at[dst_st1], rows1, sg1)
                g0.wait()
                s0 = pltpu.async_copy(rows0, agg.at[src_st0], ss0, add=True)
                g1.wait()
                s1 = pltpu.async_copy(rows1, agg.at[src_st1], ss1, add=True)
                s0.wait()
                s1.wait()
                return carry

            lax.fori_loop(0, npair, pair, 0)

            plsc.subcore_barrier()
            pltpu.sync_copy(agg.at[pl.ds(s * rpt, rpt)],
                            out_hbm.at[pl.ds(lo + s * rpt, rpt)])

    return k


def _dense(agg, h, wl_t, wc_t, bias8):
    n_pad, d = h.shape
    hid = wl_t.shape[1]
    bm = 1024

    def body(a_ref, h_ref, wl_ref, wc_ref, b_ref, o_ref):
        acc = jnp.dot(a_ref[...], wl_ref[...], preferred_element_type=jnp.float32)
        acc = acc + jnp.dot(h_ref[...], wc_ref[...],
                            preferred_element_type=jnp.float32)
        o_ref[...] = jnp.maximum(acc + b_ref[0:1, :], 0.0)

    return pl.pallas_call(
        body,
        grid=(n_pad // bm,),
        in_specs=[
            pl.BlockSpec((bm, d), lambda i: (i, 0)),
            pl.BlockSpec((bm, d), lambda i: (i, 0)),
            pl.BlockSpec((d, hid), lambda i: (0, 0)),
            pl.BlockSpec((d, hid), lambda i: (0, 0)),
            pl.BlockSpec((8, hid), lambda i: (0, 0)),
        ],
        out_specs=pl.BlockSpec((bm, hid), lambda i: (i, 0)),
        out_shape=jax.ShapeDtypeStruct((n_pad, hid), jnp.float32),
    )(agg, h, wl_t, wc_t, bias8)


def kernel(x, edge_index, edge_type, params):
    n, d0 = x.shape
    e = edge_index.shape[1]
    n_pad = ((n // 2048) + 1) * 2048
    half = n_pad // NC

    src = edge_index[0]
    dst = edge_index[1]
    h = jnp.zeros((n_pad, d0), x.dtype).at[:n].set(x)

    for rel, p in enumerate(params):
        d = h.shape[1]
        if d >= 128:
            win = 1024
        else:
            win = 2560 if half % 2560 == 0 else 1024
        agg = _make_agg(n_pad, d, e, rel, win)(h, src, dst, edge_type)
        wl_t = p["w_l_W"].T
        wc_t = (p["w_0_W"] + p["w_1_W"]).T
        bias = p["w_l_b"] + p["w_0_b"] + p["w_1_b"]
        bias8 = jnp.broadcast_to(bias[None, :], (8, bias.shape[0]))
        h = _dense(agg, h, wl_t, wc_t, bias8)

    return h[:n]


# speedup vs baseline: 4.2388x; 1.0673x over previous
import functools

import jax
import jax.numpy as jnp
from jax import lax
from jax.experimental import pallas as pl
from jax.experimental.pallas import tpu as pltpu
from jax.experimental.pallas import tpu_sc as plsc

NC = 2
NS = 16
GCH = 128
PKB = 14


def _make_agg(n_pad, d, e, rel, win):
    eps = e // NS
    cap = eps + 2 * GCH
    half = n_pad // NC
    nwin = half // win
    rpt = win // NS
    mesh = plsc.VectorSubcoreMesh(
        core_axis_name="c", subcore_axis_name="s", num_cores=NC, num_subcores=NS
    )

    @functools.partial(
        pl.kernel,
        out_type=jax.ShapeDtypeStruct((n_pad, d), jnp.float32),
        mesh=mesh,
        scratch_types=[
            pltpu.VMEM((cap,), jnp.int32),
            pltpu.VMEM((eps,), jnp.int32),
            pltpu.VMEM((eps,), jnp.int32),
            pltpu.VMEM((cap,), jnp.int32),
            pltpu.VMEM((GCH,), jnp.int32),
            pltpu.VMEM((GCH,), jnp.int32),
            pltpu.VMEM((GCH,), jnp.int32),
            pltpu.VMEM((GCH,), jnp.int32),
            pltpu.VMEM((GCH, d), jnp.float32),
            pltpu.VMEM((GCH, d), jnp.float32),
            pltpu.VMEM((rpt, d), jnp.float32),
            pltpu.VMEM_SHARED((win + 16, d), jnp.float32),
            pltpu.SemaphoreType.DMA,
            pltpu.SemaphoreType.DMA,
            pltpu.SemaphoreType.DMA,
            pltpu.SemaphoreType.DMA,
        ],
        compiler_params=pltpu.CompilerParams(needs_layout_passes=False,
                                             use_tc_tiling_on_sc=False),
    )
    def k(h_hbm, src_hbm, dst_hbm, typ_hbm, out_hbm,
          src_raw, dst_raw, typ_raw, pk1,
          src_st0, dst_st0, src_st1, dst_st1, rows0, rows1, zbuf, agg,
          sg0, sg1, ss0, ss1):
        c = lax.axis_index("c")
        s = lax.axis_index("s")
        base = s * eps

        pltpu.sync_copy(src_hbm.at[pl.ds(base, eps)], src_raw.at[pl.ds(0, eps)])
        pltpu.sync_copy(dst_hbm.at[pl.ds(base, eps)], dst_raw)
        pltpu.sync_copy(typ_hbm.at[pl.ds(base, eps)], typ_raw)

        zvec = jnp.zeros((16,), jnp.float32)

        def zrow(i, carry):
            for kk in range(d // 16):
                zbuf[i, pl.ds(kk * 16, 16)] = zvec
            return carry

        lax.fori_loop(0, rpt, zrow, 0)

        iota16 = lax.broadcasted_iota(jnp.int32, (16,), 0)

        def compact1(i, cnt):
            t = typ_raw[pl.ds(i * 16, 16)]
            sv = src_raw[pl.ds(i * 16, 16)] - c * half
            dv = dst_raw[pl.ds(i * 16, 16)]
            m = (t == rel) & (sv >= 0) & (sv < half)
            pk = lax.shift_left(sv, PKB) | dv
            inc = plsc.cumsum(m.astype(jnp.int32))
            plsc.store_scatter(pk1, [cnt + inc - 1], pk, mask=m)
            return cnt + jnp.sum(m.astype(jnp.int32))

        cnt1 = lax.fori_loop(0, eps // 16, compact1, jnp.int32(0))
        plsc.store_scatter(pk1, [cnt1 + iota16],
                           jnp.full((16,), half << PKB, jnp.int32))
        n1 = (cnt1 + 15) // 16

        pk2 = src_raw

        for w in range(nwin):
            pltpu.sync_copy(zbuf, agg.at[pl.ds(s * rpt, rpt)])

            w_lo, w_hi = w * win, (w + 1) * win

            def compact2(i, cnt):
                v = pk1[pl.ds(i * 16, 16)]
                sv = lax.shift_right_logical(v, PKB)
                m = (sv >= w_lo) & (sv < w_hi)
                inc = plsc.cumsum(m.astype(jnp.int32))
                plsc.store_scatter(pk2, [cnt + inc - 1], v, mask=m)
                return cnt + jnp.sum(m.astype(jnp.int32))

            cnt = lax.fori_loop(0, n1, compact2, jnp.int32(0))

            dummy = jnp.full((16,), w_hi << PKB, jnp.int32)
            for i in range(2 * GCH // 16):
                plsc.store_scatter(pk2, [cnt + i * 16 + iota16], dummy)

            plsc.subcore_barrier()

            def stage(off, src_st, dst_st):
                for kk in range(GCH // 16):
                    v = pk2[pl.ds(off + kk * 16, 16)]
                    src_st[pl.ds(kk * 16, 16)] = (
                        lax.shift_right_logical(v, PKB) - w_lo)
                    dst_st[pl.ds(kk * 16, 16)] = v & ((1 << PKB) - 1)

            npair = (cnt + 2 * GCH - 1) // (2 * GCH)

            def pair(kk, carry):
                off = kk * (2 * GCH)
                stage(off, src_st0, dst_st0)
                g0 = pltpu.async_copy(h_hbm.at[dst_st0], rows0, sg0)
                stage(off + GCH, src_st1, dst_st1)
                g1 = pltpu.async_copy(h_hbm.at[dst_st1], rows1, sg1)
                g0.wait()
                s0 = pltpu.async_copy(rows0, agg.at[src_st0], ss0, add=True)
                g1.wait()
                s1 = pltpu.async_copy(rows1, agg.at[src_st1], ss1, add=True)
                s0.wait()
                s1.wait()
                return carry

            lax.fori_loop(0, npair, pair, 0)

            plsc.subcore_barrier()
            pltpu.sync_copy(agg.at[pl.ds(s * rpt, rpt)],
                            out_hbm.at[pl.ds(c * half + w_lo + s * rpt, rpt)])

    return k


def _dense(agg, h, wl_t, wc_t, bias8):
    n_pad, d = h.shape
    hid = wl_t.shape[1]
    bm = 1024

    def body(a_ref, h_ref, wl_ref, wc_ref, b_ref, o_ref):
        acc = jnp.dot(a_ref[...], wl_ref[...], preferred_element_type=jnp.float32)
        acc = acc + jnp.dot(h_ref[...], wc_ref[...],
                            preferred_element_type=jnp.float32)
        o_ref[...] = jnp.maximum(acc + b_ref[0:1, :], 0.0)

    return pl.pallas_call(
        body,
        grid=(n_pad // bm,),
        in_specs=[
            pl.BlockSpec((bm, d), lambda i: (i, 0)),
            pl.BlockSpec((bm, d), lambda i: (i, 0)),
            pl.BlockSpec((d, hid), lambda i: (0, 0)),
            pl.BlockSpec((d, hid), lambda i: (0, 0)),
            pl.BlockSpec((8, hid), lambda i: (0, 0)),
        ],
        out_specs=pl.BlockSpec((bm, hid), lambda i: (i, 0)),
        out_shape=jax.ShapeDtypeStruct((n_pad, hid), jnp.float32),
    )(agg, h, wl_t, wc_t, bias8)


def kernel(x, edge_index, edge_type, params):
    n, d0 = x.shape
    e = edge_index.shape[1]
    n_pad = ((n // 2048) + 1) * 2048
    half = n_pad // NC

    src = edge_index[0]
    dst = edge_index[1]
    h = jnp.zeros((n_pad, d0), x.dtype).at[:n].set(x)

    for rel, p in enumerate(params):
        d = h.shape[1]
        if d >= 128:
            win = 1024
        else:
            win = 2560 if half % 2560 == 0 else 1024
        agg = _make_agg(n_pad, d, e, rel, win)(h, src, dst, edge_type)
        wl_t = p["w_l_W"].T
        wc_t = (p["w_0_W"] + p["w_1_W"]).T
        bias = p["w_l_b"] + p["w_0_b"] + p["w_1_b"]
        bias8 = jnp.broadcast_to(bias[None, :], (8, bias.shape[0]))
        h = _dense(agg, h, wl_t, wc_t, bias8)

    return h[:n]
